# Initial kernel scaffold; baseline (speedup 1.0000x reference)
import jax
import jax.numpy as jnp
from jax.experimental import pallas as pl


def kernel(x_transaction, x_merchant, x_category, ei_belongs_to, ei_has_category, ei_rev_belongs_to, ei_rev_has_category, ei_self_transaction, ei_self_merchant, ei_self_category, W_enc_t, b_enc_t, W_enc_m, b_enc_m, W_enc_c, b_enc_c, W_l, b_l, W_r, W_pre, b_pre, W_cls, b_cls):
    raise NotImplementedError("write your pallas kernel here")


# speedup vs baseline: 1.3822x; 1.3822x over previous
import functools

import jax
import jax.numpy as jnp
from jax.experimental import pallas as pl

_NT, _NM, _NC = 100000, 5000, 400
_H = 128
_BLK = 1000


def _enc_kernel(x_ref, w_ref, b_ref, o_ref):
    o_ref[...] = jnp.maximum(
        jnp.dot(x_ref[...], w_ref[...], preferred_element_type=jnp.float32)
        + b_ref[...],
        0.0,
    )


def _encode(x, w, b, blk):
    n, din = x.shape
    h = w.shape[1]
    grid = n // blk
    return pl.pallas_call(
        _enc_kernel,
        grid=(grid,),
        in_specs=[
            pl.BlockSpec((blk, din), lambda i: (i, 0)),
            pl.BlockSpec((din, h), lambda i: (0, 0)),
            pl.BlockSpec((h,), lambda i: (0,)),
        ],
        out_specs=pl.BlockSpec((blk, h), lambda i: (i, 0)),
        out_shape=jax.ShapeDtypeStruct((n, h), jnp.float32),
    )(x, w, b)


def _combine2_kernel(s2_ref, c2_ref, s3_ref, c3_ref, x_ref, res_ref,
                     wl2_ref, wl3_ref, wc_ref, b_ref, o_ref):
    m2 = s2_ref[...] / jnp.maximum(c2_ref[...], 1.0)
    m3 = s3_ref[...] / jnp.maximum(c3_ref[...], 1.0)
    acc = jnp.dot(m2, wl2_ref[...], preferred_element_type=jnp.float32)
    acc += jnp.dot(m3, wl3_ref[...], preferred_element_type=jnp.float32)
    acc += jnp.dot(x_ref[...], wc_ref[...], preferred_element_type=jnp.float32)
    acc += b_ref[...] + res_ref[...]
    o_ref[...] = jnp.maximum(acc, 0.0)


def _combine2(s2, c2, s3, c3, x, res, wl2, wl3, wc, b, blk):
    n = x.shape[0]
    grid = n // blk
    spec_r = pl.BlockSpec((blk, _H), lambda i: (i, 0))
    spec_c = pl.BlockSpec((blk, 1), lambda i: (i, 0))
    spec_w = pl.BlockSpec((_H, _H), lambda i: (0, 0))
    return pl.pallas_call(
        _combine2_kernel,
        grid=(grid,),
        in_specs=[spec_r, spec_c, spec_r, spec_c, spec_r, spec_r,
                  spec_w, spec_w, spec_w, pl.BlockSpec((_H,), lambda i: (0,))],
        out_specs=spec_r,
        out_shape=jax.ShapeDtypeStruct((n, _H), jnp.float32),
    )(s2, c2, s3, c3, x, res, wl2, wl3, wc, b)


def _combine1_kernel(s_ref, c_ref, x_ref, res_ref, wl_ref, wc_ref, b_ref,
                     o_ref):
    m = s_ref[...] / jnp.maximum(c_ref[...], 1.0)
    acc = jnp.dot(m, wl_ref[...], preferred_element_type=jnp.float32)
    acc += jnp.dot(x_ref[...], wc_ref[...], preferred_element_type=jnp.float32)
    acc += b_ref[...] + res_ref[...]
    o_ref[...] = jnp.maximum(acc, 0.0)


def _combine1(s, c, x, res, wl, wc, b, blk):
    n = x.shape[0]
    grid = n // blk
    spec_r = pl.BlockSpec((blk, _H), lambda i: (i, 0))
    spec_c = pl.BlockSpec((blk, 1), lambda i: (i, 0))
    spec_w = pl.BlockSpec((_H, _H), lambda i: (0, 0))
    return pl.pallas_call(
        _combine1_kernel,
        grid=(grid,),
        in_specs=[spec_r, spec_c, spec_r, spec_r, spec_w, spec_w,
                  pl.BlockSpec((_H,), lambda i: (0,))],
        out_specs=spec_r,
        out_shape=jax.ShapeDtypeStruct((n, _H), jnp.float32),
    )(s, c, x, res, wl, wc, b)


def _head_kernel(x_ref, wp_ref, bp_ref, wc_ref, bc_ref, o_ref):
    h = jnp.maximum(
        jnp.dot(x_ref[...], wp_ref[...], preferred_element_type=jnp.float32)
        + bp_ref[...],
        0.0,
    )
    o_ref[...] = (
        jnp.dot(h, wc_ref[...], preferred_element_type=jnp.float32)
        + bc_ref[...]
    )


def _head(x, wp, bp, wc, bc, blk):
    n = x.shape[0]
    ncls = wc.shape[1]
    grid = n // blk
    return pl.pallas_call(
        _head_kernel,
        grid=(grid,),
        in_specs=[
            pl.BlockSpec((blk, _H), lambda i: (i, 0)),
            pl.BlockSpec((_H, _H), lambda i: (0, 0)),
            pl.BlockSpec((_H,), lambda i: (0,)),
            pl.BlockSpec((_H, ncls), lambda i: (0, 0)),
            pl.BlockSpec((ncls,), lambda i: (0,)),
        ],
        out_specs=pl.BlockSpec((blk, ncls), lambda i: (i, 0)),
        out_shape=jax.ShapeDtypeStruct((n, ncls), jnp.float32),
    )(x, wp, bp, wc, bc)


def _segsum(x_src, src, dst, n_dst):
    msg = jnp.take(x_src, src, axis=0)
    return jax.ops.segment_sum(msg, dst, num_segments=n_dst)


def _counts(dst, e, n_dst):
    return jax.ops.segment_sum(
        jnp.ones((e, 1), jnp.float32), dst, num_segments=n_dst
    )


@jax.jit
def kernel(x_transaction, x_merchant, x_category, ei_belongs_to,
           ei_has_category, ei_rev_belongs_to, ei_rev_has_category,
           ei_self_transaction, ei_self_merchant, ei_self_category,
           W_enc_t, b_enc_t, W_enc_m, b_enc_m, W_enc_c, b_enc_c,
           W_l, b_l, W_r, W_pre, b_pre, W_cls, b_cls):
    xt = _encode(x_transaction, W_enc_t, b_enc_t, _BLK)
    xm = _encode(x_merchant, W_enc_m, b_enc_m, 1000)
    xc = _encode(x_category, W_enc_c, b_enc_c, 400)

    e0 = ei_belongs_to.shape[1]
    cnt_m = _counts(ei_belongs_to[1], e0, _NM)
    cnt_c = _counts(ei_has_category[1], ei_has_category.shape[1], _NC)
    cnt_t2 = _counts(ei_rev_belongs_to[1], ei_rev_belongs_to.shape[1], _NT)
    cnt_t3 = _counts(ei_rev_has_category[1], ei_rev_has_category.shape[1],
                     _NT)

    for i in range(3):
        wc_t = W_r[i, 2] + W_r[i, 3] + W_l[i, 4] + W_r[i, 4]
        b_t = b_l[i, 2] + b_l[i, 3] + b_l[i, 4]
        wc_m = W_r[i, 0] + W_l[i, 5] + W_r[i, 5]
        b_m = b_l[i, 0] + b_l[i, 5]
        wc_c = W_r[i, 1] + W_l[i, 6] + W_r[i, 6]
        b_c = b_l[i, 1] + b_l[i, 6]

        s_m = _segsum(xt, ei_belongs_to[0], ei_belongs_to[1], _NM)
        s_c = _segsum(xt, ei_has_category[0], ei_has_category[1], _NC)
        s_t2 = _segsum(xm, ei_rev_belongs_to[0], ei_rev_belongs_to[1], _NT)
        s_t3 = _segsum(xc, ei_rev_has_category[0], ei_rev_has_category[1],
                       _NT)

        res_t = xt if i > 0 else jnp.zeros_like(xt)
        res_m = xm if i > 0 else jnp.zeros_like(xm)
        res_c = xc if i > 0 else jnp.zeros_like(xc)

        xt_new = _combine2(s_t2, cnt_t2, s_t3, cnt_t3, xt, res_t,
                           W_l[i, 2], W_l[i, 3], wc_t, b_t, _BLK)
        xm_new = _combine1(s_m, cnt_m, xm, res_m, W_l[i, 0], wc_m, b_m, 1000)
        xc_new = _combine1(s_c, cnt_c, xc, res_c, W_l[i, 1], wc_c, b_c, 400)
        xt, xm, xc = xt_new, xm_new, xc_new

    return _head(xt, W_pre, b_pre, W_cls, b_cls, _BLK)
